# Initial kernel scaffold; baseline (speedup 1.0000x reference)
import jax
import jax.numpy as jnp
from jax.experimental import pallas as pl


def kernel(x, P_w, U1, U2, U3, W1, W2, b2):
    raise NotImplementedError("write your pallas kernel here")


# speedup vs baseline: 2.7498x; 2.7498x over previous
import functools

import jax
import jax.numpy as jnp
from jax.experimental import pallas as pl
from jax.experimental.pallas import tpu as pltpu

_HIGH = jax.lax.Precision.HIGHEST


def _top2_coeff(z, tau):
    idx = jax.lax.broadcasted_iota(jnp.int32, z.shape, 1)
    v1 = jnp.max(z, axis=1, keepdims=True)
    i1 = jnp.min(jnp.where(z == v1, idx, z.shape[1]), axis=1, keepdims=True)
    m1 = idx == i1
    z2 = jnp.where(m1, -jnp.inf, z)
    v2 = jnp.max(z2, axis=1, keepdims=True)
    i2 = jnp.min(jnp.where(z2 == v2, idx, z.shape[1]), axis=1, keepdims=True)
    m2 = idx == i2
    t = tau + 1e-8
    a = jnp.exp((v2 - v1) / t)
    w1 = 1.0 / (1.0 + a)
    w2 = a / (1.0 + a)
    return jnp.where(m1, w1, 0.0) + jnp.where(m2, w2, 0.0)


def _fused_body(x_ref, pw_ref, u_ref, w1_ref, w2_ref, b2_ref, out_ref):
    xt = x_ref[...]
    xb = xt.astype(jnp.bfloat16)
    addr = jax.lax.dot_general(xb, pw_ref[...], (((1,), (1,)), ((), ())),
                               preferred_element_type=jnp.float32)
    zz = jax.lax.dot_general(addr.astype(jnp.bfloat16), u_ref[...],
                             (((1,), (1,)), ((), ())),
                             preferred_element_type=jnp.float32)
    c1 = _top2_coeff(zz[:, 0:8], 1.0)
    c2 = _top2_coeff(zz[:, 8:16], 1.0)
    c3 = _top2_coeff(zz[:, 16:24], 1.0)

    h = None
    for l in range(8):
        yl = jax.lax.dot_general(xb, w1_ref[l], (((1,), (1,)), ((), ())),
                                 preferred_element_type=jnp.float32)
        h = yl * c1[:, l:l + 1] if h is None else h + yl * c1[:, l:l + 1]
    h = 0.5 * h * (1.0 + jax.lax.erf(h * 0.7071067811865476))

    hb = h.astype(jnp.bfloat16)
    y = jax.lax.dot_general(c3, b2_ref[...], (((1,), (0,)), ((), ())),
                            precision=_HIGH,
                            preferred_element_type=jnp.float32)
    for l in range(8):
        yl = jax.lax.dot_general(hb, w2_ref[l], (((1,), (1,)), ((), ())),
                                 preferred_element_type=jnp.float32)
        y = y + yl * c2[:, l:l + 1]
    out_ref[...] = y


@functools.partial(jax.jit, static_argnames=())
def kernel(x, P_w, U1, U2, U3, W1, W2, b2):
    Bx, Tx, D = x.shape
    N = Bx * Tx
    H = W1.shape[1]
    DO = W2.shape[1]
    x_flat = x.reshape(N, D)
    u_pack = jnp.concatenate([U1, U2, U3], axis=0).astype(jnp.bfloat16)
    pwb = P_w.astype(jnp.bfloat16)
    w1b = W1.astype(jnp.bfloat16)
    w2b = W2.astype(jnp.bfloat16)

    TM = 256
    grid = (N // TM,)
    out = pl.pallas_call(
        _fused_body,
        grid=grid,
        in_specs=[
            pl.BlockSpec((TM, D), lambda i: (i, 0)),
            pl.BlockSpec(pwb.shape, lambda i: (0, 0)),
            pl.BlockSpec(u_pack.shape, lambda i: (0, 0)),
            pl.BlockSpec(w1b.shape, lambda i: (0, 0, 0)),
            pl.BlockSpec(w2b.shape, lambda i: (0, 0, 0)),
            pl.BlockSpec(b2.shape, lambda i: (0, 0)),
        ],
        out_specs=pl.BlockSpec((TM, DO), lambda i: (i, 0)),
        out_shape=jax.ShapeDtypeStruct((N, DO), jnp.float32),
    )(x_flat, pwb, u_pack, w1b, w2b, b2)
    return out.reshape(Bx, Tx, DO)
